# baseline (speedup 1.0000x reference)
import jax
import jax.numpy as jnp
from jax import lax
from jax.experimental import pallas as pl
from jax.experimental.pallas import tpu as pltpu
from jax.experimental.pallas import tpu_sc as plsc

NUM_LAYERS = 2
P = 512
NLEV = 16
OUT = 4096
SCALE = OUT // P
SLICING = 0.001

SC_CORES = 2
SC_SUBCORES = 16
SC_WORKERS = SC_CORES * SC_SUBCORES
ROWS_PER_TILE = P // SC_WORKERS
LANES = 16
CHUNKS = P // LANES


def _levels_body(logits_ref, u_ref, out_ref):
    lvl = lax.broadcasted_iota(jnp.int32, (P, NLEV), 1).astype(jnp.float32)
    for l in range(NUM_LAYERS):
        u = u_ref[l]
        g = -jnp.log(-jnp.log(u + 1e-20) + 1e-20)
        x = logits_ref[l] + g
        m = jnp.max(x, axis=1, keepdims=True)
        e = jnp.exp(x - m)
        s = jnp.sum(e, axis=1)
        w = jnp.sum(e * lvl, axis=1)
        out_ref[l, :] = w / s * SLICING


_levels_call = pl.pallas_call(
    _levels_body,
    out_shape=jax.ShapeDtypeStruct((NUM_LAYERS, P), jnp.float32),
)


def _gather_body(tab_hbm, idx_hbm, out_hbm, tab_v, idx_v, out_v):
    wid = lax.axis_index("s") * SC_CORES + lax.axis_index("c")
    rbase = wid * ROWS_PER_TILE
    pltpu.sync_copy(tab_hbm, tab_v)
    pltpu.sync_copy(idx_hbm.at[pl.ds(rbase, ROWS_PER_TILE)], idx_v)

    @plsc.parallel_loop(0, ROWS_PER_TILE, unroll=2)
    def row(j):
        for c in range(CHUNKS):
            iv = idx_v[j, pl.ds(c * LANES, LANES)]
            out_v[j, pl.ds(c * LANES, LANES)] = plsc.load_gather(tab_v, [iv])

    pltpu.sync_copy(out_v, out_hbm.at[pl.ds(rbase, ROWS_PER_TILE)])


_gather_call_cache = []


def _gather_call(tab_l, indices):
    if not _gather_call_cache:
        _gather_call_cache.append(pl.kernel(
            _gather_body,
            out_type=jax.ShapeDtypeStruct((P, P), jnp.float32),
            mesh=plsc.VectorSubcoreMesh(core_axis_name="c", subcore_axis_name="s"),
            compiler_params=pltpu.CompilerParams(needs_layout_passes=False),
            scratch_types=[
                pltpu.VMEM((P,), jnp.float32),
                pltpu.VMEM((ROWS_PER_TILE, P), jnp.int32),
                pltpu.VMEM((ROWS_PER_TILE, P), jnp.float32),
            ],
        ))
    return _gather_call_cache[0](tab_l, indices)


ROWS_BLK = P // SCALE


def _build_r(r_ref):
    cols = lax.broadcasted_iota(jnp.int32, (P, OUT), 1)
    rows = lax.broadcasted_iota(jnp.int32, (P, OUT), 0)
    r_ref[...] = ((cols // SCALE) == rows).astype(jnp.bfloat16)


def _expand(g, r_ref):
    w = jnp.dot(g.astype(jnp.bfloat16), r_ref[...],
                preferred_element_type=jnp.float32)
    return jnp.broadcast_to(w[:, None, :], (ROWS_BLK, SCALE, OUT)).reshape(P, OUT)


def _up0_body(g_ref, out_ref, r_ref):
    @pl.when(pl.program_id(0) == 0)
    def _():
        _build_r(r_ref)
    out_ref[0] = _expand(g_ref[...], r_ref)


def _up1_body(big_ref, g_ref, out_ref, r_ref):
    del big_ref
    @pl.when(pl.program_id(0) == 0)
    def _():
        _build_r(r_ref)
    out_ref[0] = _expand(g_ref[...], r_ref)


_up0_call = pl.pallas_call(
    _up0_body,
    grid=(OUT // P,),
    in_specs=[pl.BlockSpec((ROWS_BLK, P), lambda k: (k, 0))],
    out_specs=pl.BlockSpec((1, P, OUT), lambda k: (0, k, 0)),
    out_shape=jax.ShapeDtypeStruct((NUM_LAYERS, OUT, OUT), jnp.float32),
    scratch_shapes=[pltpu.VMEM((P, OUT), jnp.bfloat16)],
)

_up1_call = pl.pallas_call(
    _up1_body,
    grid=(OUT // P,),
    in_specs=[
        pl.BlockSpec(memory_space=pl.ANY),
        pl.BlockSpec((ROWS_BLK, P), lambda k: (k, 0)),
    ],
    out_specs=pl.BlockSpec((1, P, OUT), lambda k: (1, k, 0)),
    out_shape=jax.ShapeDtypeStruct((NUM_LAYERS, OUT, OUT), jnp.float32),
    input_output_aliases={0: 0},
    scratch_shapes=[pltpu.VMEM((P, OUT), jnp.bfloat16)],
)


def kernel(logits, u, indices):
    tab = _levels_call(logits, u)
    g0 = _gather_call(tab[0], indices)
    g1 = _gather_call(tab[1], indices)
    big = _up0_call(g0)
    big = _up1_call(big, g1)
    return big[None]
